# Initial kernel scaffold; baseline (speedup 1.0000x reference)
import jax
import jax.numpy as jnp
from jax.experimental import pallas as pl


def kernel(pillar_features, coords):
    raise NotImplementedError("write your pallas kernel here")


# speedup vs baseline: 1.0718x; 1.0718x over previous
import functools

import jax
import jax.numpy as jnp
from jax import lax
from jax.experimental import pallas as pl
from jax.experimental.pallas import tpu as pltpu
from jax.experimental.pallas import tpu_sc as plsc

_C = 64
_NX = 512
_NY = 512
_NXY = _NX * _NY
_P = 30000
_PPAD = 4096
_NC = 2
_NS = 16
_NW = _NC * _NS
_COLS_PER_W = _NXY // _NW
_CHUNK = 1024
_CCHUNK = 2000
_L = 16


def _sc_scatter(coords, pfpad):
    mesh = plsc.VectorSubcoreMesh(core_axis_name="c", subcore_axis_name="s")

    @functools.partial(
        pl.kernel,
        mesh=mesh,
        compiler_params=pltpu.CompilerParams(
            needs_layout_passes=False, use_tc_tiling_on_sc=False),
        out_type=jax.ShapeDtypeStruct((_NXY, _C), jnp.float32),
        scratch_types=[
            pltpu.VMEM((_COLS_PER_W,), jnp.int32),
            pltpu.VMEM((_CCHUNK * 4,), jnp.int32),
            pltpu.VMEM((_CHUNK,), jnp.int32),
            pltpu.VMEM((_CHUNK, _C), jnp.float32),
            pltpu.SemaphoreType.DMA,
        ],
    )
    def k(coords_hbm, pf_hbm, out_hbm, win_v, crd_v, plist_v, rows_v, sem):
        wid = lax.axis_index("s") * _NC + lax.axis_index("c")
        lo = wid * _COLS_PER_W
        lane = jnp.arange(_L, dtype=jnp.int32)

        def init_body(i, _):
            win_v[pl.ds(i * _L, _L)] = jnp.full((_L,), -1, jnp.int32)
            return 0

        lax.fori_loop(0, _COLS_PER_W // _L, init_body, 0)

        def chunk_body(ci, _):
            pltpu.sync_copy(
                coords_hbm.at[pl.ds(ci * _CCHUNK * 4, _CCHUNK * 4)], crd_v)

            def vec_body(vi, _):
                r16 = vi * _L + lane
                c2 = plsc.load_gather(crd_v, [r16 * 4 + 2])
                c3 = plsc.load_gather(crd_v, [r16 * 4 + 3])
                idx = c2 * _NX + c3
                p = ci * _CCHUNK + r16
                own = (idx >= lo) & (idx < lo + _COLS_PER_W)
                idxl = jnp.where(own, idx - lo, 0)
                key = (idxl << 15) | p

                def rmw(_go):
                    cur = plsc.load_gather(win_v, [idxl], mask=own)
                    plsc.store_scatter(win_v, [idxl], jnp.maximum(cur, key),
                                       mask=own)
                    chk = plsc.load_gather(win_v, [idxl], mask=own)
                    return jnp.any(own & (chk < key))

                lax.while_loop(lambda g: g, rmw, jnp.bool_(True))
                return 0

            lax.fori_loop(0, _CCHUNK // _L, vec_body, 0)
            return 0

        lax.fori_loop(0, _P // _CCHUNK, chunk_body, 0)

        def out_chunk(kk, _):
            cbase = kk * _CHUNK

            def build(vi, _):
                w = win_v[pl.ds(cbase + vi * _L, _L)]
                pwin = w & 32767
                col16 = cbase + vi * _L + lane
                dummy = _P + (col16 & (_PPAD - 1))
                plist_v[pl.ds(vi * _L, _L)] = jnp.where(w >= 0, pwin, dummy)
                return 0

            lax.fori_loop(0, _CHUNK // _L, build, 0)
            pltpu.async_copy(pf_hbm.at[plist_v], rows_v, sem).wait()
            pltpu.sync_copy(rows_v, out_hbm.at[pl.ds(lo + cbase, _CHUNK)])
            return 0

        lax.fori_loop(0, _COLS_PER_W // _CHUNK, out_chunk, 0)

    return k(coords, pfpad)


def kernel(pillar_features, coords):
    pfpad = jnp.concatenate(
        [pillar_features, jnp.zeros((_PPAD, _C), jnp.float32)], axis=0)
    out_t = _sc_scatter(coords.astype(jnp.int32).reshape(-1), pfpad)
    return out_t.T.reshape(1, _C, _NY, _NX)
